# baseline (speedup 1.0000x reference)
import functools

import jax
import jax.numpy as jnp
from jax import lax
from jax.experimental import pallas as pl
from jax.experimental.pallas import tpu as pltpu
from jax.experimental.pallas import tpu_sc as plsc

_NC = 2
_NS = 16
_NW = _NC * _NS

_B = 4 * 2048
_D = 1280
_S = 2048
_BPW = _S // _NW
_CHUNK = 32
_NCHUNK = _BPW // _CHUNK
_NBUF = 3

_V = 64
_R = 2048


def _table_body(emb_ref, w_ref, b_ref, out_ref):
    w_sum = w_ref[:, :512] + w_ref[:, 512:]
    acc = jax.lax.dot_general(
        emb_ref[:], w_sum,
        dimension_numbers=(((1,), (1,)), ((), ())),
        preferred_element_type=jnp.float32,
    )
    out_ref[:] = acc + b_ref[:]


def _compute_table(emb, w, b):
    emb_pad = jnp.zeros((_V, 512), jnp.float32).at[:33].set(emb)
    return pl.pallas_call(
        _table_body,
        out_shape=jax.ShapeDtypeStruct((_V, _D), jnp.float32),
    )(emb_pad, w, b.reshape(1, _D))


def _gather_body(tok_hbm, table_hbm, out_hbm, idx_v, bufs, gsems, wsems):
    wid = lax.axis_index("s") * _NC + lax.axis_index("c")
    base = wid * _BPW
    pltpu.sync_copy(tok_hbm.at[wid], idx_v)

    def issue(c):
        s = c % _NBUF
        return pltpu.async_copy(
            table_hbm.at[idx_v.at[c]], bufs[s], gsems[s])

    gathers = [None] * _NBUF
    writes = [None] * _NBUF
    for c in range(min(_NBUF - 1, _NCHUNK)):
        gathers[c % _NBUF] = issue(c)
    for c in range(_NCHUNK):
        s = c % _NBUF
        if gathers[s] is None:
            gathers[s] = issue(c)
        gathers[s].wait()
        writes[s] = pltpu.async_copy(
            bufs[s], out_hbm.at[pl.ds(base + c * _CHUNK, _CHUNK)], wsems[s])
        n = c + _NBUF - 1
        if n < _NCHUNK:
            s2 = n % _NBUF
            if writes[s2] is not None:
                writes[s2].wait()
            gathers[s2] = issue(n)
    for w in writes:
        if w is not None:
            w.wait()


_gather = functools.partial(
    pl.kernel,
    out_type=jax.ShapeDtypeStruct((_B, _D), jnp.float32),
    mesh=plsc.VectorSubcoreMesh(
        core_axis_name="c", subcore_axis_name="s",
        num_cores=_NC, num_subcores=_NS),
    scratch_types=[
        pltpu.VMEM((_NCHUNK, _CHUNK), jnp.int32),
        [pltpu.VMEM((_CHUNK, _D), jnp.float32) for _ in range(_NBUF)],
        [pltpu.SemaphoreType.DMA for _ in range(_NBUF)],
        [pltpu.SemaphoreType.DMA for _ in range(_NBUF)],
    ],
)(_gather_body)


def _expand_body(tok_ref, table_ref, alias_ref, out_ref):
    del alias_ref
    tok = tok_ref[0]
    oh = (tok[0, :, None] == lax.broadcasted_iota(jnp.int32, (1, _V), 1)
          ).astype(jnp.float32)
    out_ref[:] = jax.lax.dot_general(
        oh, table_ref[:],
        dimension_numbers=(((1,), (0,)), ((), ())),
        preferred_element_type=jnp.float32)


_expand = pl.pallas_call(
    _expand_body,
    grid=((_B - _S) // _R,),
    in_specs=[
        pl.BlockSpec((1, 1, _R), lambda i: (i, 0, 0)),
        pl.BlockSpec((_V, _D), lambda i: (0, 0)),
        pl.BlockSpec(memory_space=pl.ANY),
    ],
    out_specs=pl.BlockSpec((_R, _D), lambda i: (_S // _R + i, 0)),
    out_shape=jax.ShapeDtypeStruct((_B, _D), jnp.float32),
    input_output_aliases={2: 0},
)


def kernel(tokens, emb, W, b):
    table = _compute_table(emb, W, b)
    tok = tokens.astype(jnp.int32).reshape(-1)
    tok_lo = tok[:_S].reshape(_NW, _NCHUNK, _CHUNK)
    tok_hi = tok[_S:].reshape((_B - _S) // _R, 1, _R)
    partial = _gather(tok_lo, table)
    out = _expand(tok_hi, table, partial)
    return out.reshape(tokens.shape[0], tokens.shape[1], _D)


# speedup vs baseline: 1.0183x; 1.0183x over previous
import functools

import jax
import jax.numpy as jnp
from jax import lax
from jax.experimental import pallas as pl
from jax.experimental.pallas import tpu as pltpu
from jax.experimental.pallas import tpu_sc as plsc

_NC = 2
_NS = 16
_NW = _NC * _NS

_B = 4 * 2048
_D = 1280
_S = 2048
_BPW = _S // _NW
_CHUNK = 32
_NCHUNK = _BPW // _CHUNK
_NBUF = 3

_V = 64
_R = 1024


def _table_body(emb_ref, w_ref, b_ref, out_ref):
    w_sum = w_ref[:, :512] + w_ref[:, 512:]
    acc = jax.lax.dot_general(
        emb_ref[:], w_sum,
        dimension_numbers=(((1,), (1,)), ((), ())),
        preferred_element_type=jnp.float32,
    )
    out_ref[:] = acc + b_ref[:]


def _compute_table(emb, w, b):
    emb_pad = jnp.zeros((_V, 512), jnp.float32).at[:33].set(emb)
    return pl.pallas_call(
        _table_body,
        out_shape=jax.ShapeDtypeStruct((_V, _D), jnp.float32),
    )(emb_pad, w, b.reshape(1, _D))


def _gather_body(tok_hbm, table_hbm, out_hbm, idx_v, bufs, gsems, wsems):
    wid = lax.axis_index("s") * _NC + lax.axis_index("c")
    base = wid * _BPW
    pltpu.sync_copy(tok_hbm.at[wid], idx_v)

    def issue(c):
        s = c % _NBUF
        return pltpu.async_copy(
            table_hbm.at[idx_v.at[c]], bufs[s], gsems[s])

    gathers = [None] * _NBUF
    writes = [None] * _NBUF
    for c in range(min(_NBUF - 1, _NCHUNK)):
        gathers[c % _NBUF] = issue(c)
    for c in range(_NCHUNK):
        s = c % _NBUF
        if gathers[s] is None:
            gathers[s] = issue(c)
        gathers[s].wait()
        writes[s] = pltpu.async_copy(
            bufs[s], out_hbm.at[pl.ds(base + c * _CHUNK, _CHUNK)], wsems[s])
        n = c + _NBUF - 1
        if n < _NCHUNK:
            s2 = n % _NBUF
            if writes[s2] is not None:
                writes[s2].wait()
            gathers[s2] = issue(n)
    for w in writes:
        if w is not None:
            w.wait()


_gather = functools.partial(
    pl.kernel,
    out_type=jax.ShapeDtypeStruct((_B, _D), jnp.float32),
    mesh=plsc.VectorSubcoreMesh(
        core_axis_name="c", subcore_axis_name="s",
        num_cores=_NC, num_subcores=_NS),
    scratch_types=[
        pltpu.VMEM((_NCHUNK, _CHUNK), jnp.int32),
        [pltpu.VMEM((_CHUNK, _D), jnp.float32) for _ in range(_NBUF)],
        [pltpu.SemaphoreType.DMA for _ in range(_NBUF)],
        [pltpu.SemaphoreType.DMA for _ in range(_NBUF)],
    ],
)(_gather_body)


def _expand_body(tok_ref, table_ref, alias_ref, out_ref):
    del alias_ref
    tok = tok_ref[0]
    oh = (tok[0, :, None] == lax.broadcasted_iota(jnp.int32, (1, _V), 1)
          ).astype(jnp.float32)
    out_ref[:] = jax.lax.dot_general(
        oh, table_ref[:],
        dimension_numbers=(((1,), (0,)), ((), ())),
        preferred_element_type=jnp.float32)


_expand = pl.pallas_call(
    _expand_body,
    grid=((_B - _S) // _R,),
    in_specs=[
        pl.BlockSpec((1, 1, _R), lambda i: (i, 0, 0)),
        pl.BlockSpec((_V, _D), lambda i: (0, 0)),
        pl.BlockSpec(memory_space=pl.ANY),
    ],
    out_specs=pl.BlockSpec((_R, _D), lambda i: (_S // _R + i, 0)),
    out_shape=jax.ShapeDtypeStruct((_B, _D), jnp.float32),
    input_output_aliases={2: 0},
)


def kernel(tokens, emb, W, b):
    table = _compute_table(emb, W, b)
    tok = tokens.astype(jnp.int32).reshape(-1)
    tok_lo = tok[:_S].reshape(_NW, _NCHUNK, _CHUNK)
    tok_hi = tok[_S:].reshape((_B - _S) // _R, 1, _R)
    partial = _gather(tok_lo, table)
    out = _expand(tok_hi, table, partial)
    return out.reshape(tokens.shape[0], tokens.shape[1], _D)


# speedup vs baseline: 1.0634x; 1.0443x over previous
import functools

import jax
import jax.numpy as jnp
from jax import lax
from jax.experimental import pallas as pl
from jax.experimental.pallas import tpu as pltpu
from jax.experimental.pallas import tpu_sc as plsc

_NC = 2
_NS = 16
_NW = _NC * _NS

_B = 4 * 2048
_D = 1280
_S = 2048
_BPW = _S // _NW
_CHUNK = 16
_NCHUNK = _BPW // _CHUNK
_NBUF = 6

_V = 64
_R = 1024


def _table_body(emb_ref, w_ref, b_ref, out_ref):
    w_sum = w_ref[:, :512] + w_ref[:, 512:]
    acc = jax.lax.dot_general(
        emb_ref[:], w_sum,
        dimension_numbers=(((1,), (1,)), ((), ())),
        preferred_element_type=jnp.float32,
    )
    out_ref[:] = acc + b_ref[:]


def _compute_table(emb, w, b):
    emb_pad = jnp.zeros((_V, 512), jnp.float32).at[:33].set(emb)
    return pl.pallas_call(
        _table_body,
        out_shape=jax.ShapeDtypeStruct((_V, _D), jnp.float32),
    )(emb_pad, w, b.reshape(1, _D))


def _gather_body(tok_hbm, table_hbm, out_hbm, idx_v, bufs, gsems, wsems):
    wid = lax.axis_index("s") * _NC + lax.axis_index("c")
    base = wid * _BPW
    pltpu.sync_copy(tok_hbm.at[wid], idx_v)

    def issue(c):
        s = c % _NBUF
        return pltpu.async_copy(
            table_hbm.at[idx_v.at[c]], bufs[s], gsems[s])

    gathers = [None] * _NBUF
    writes = [None] * _NBUF
    for c in range(min(_NBUF - 1, _NCHUNK)):
        gathers[c % _NBUF] = issue(c)
    for c in range(_NCHUNK):
        s = c % _NBUF
        if gathers[s] is None:
            gathers[s] = issue(c)
        gathers[s].wait()
        writes[s] = pltpu.async_copy(
            bufs[s], out_hbm.at[pl.ds(base + c * _CHUNK, _CHUNK)], wsems[s])
        n = c + _NBUF - 1
        if n < _NCHUNK:
            s2 = n % _NBUF
            if writes[s2] is not None:
                writes[s2].wait()
            gathers[s2] = issue(n)
    for w in writes:
        if w is not None:
            w.wait()


_gather = functools.partial(
    pl.kernel,
    out_type=jax.ShapeDtypeStruct((_B, _D), jnp.float32),
    mesh=plsc.VectorSubcoreMesh(
        core_axis_name="c", subcore_axis_name="s",
        num_cores=_NC, num_subcores=_NS),
    scratch_types=[
        pltpu.VMEM((_NCHUNK, _CHUNK), jnp.int32),
        [pltpu.VMEM((_CHUNK, _D), jnp.float32) for _ in range(_NBUF)],
        [pltpu.SemaphoreType.DMA for _ in range(_NBUF)],
        [pltpu.SemaphoreType.DMA for _ in range(_NBUF)],
    ],
)(_gather_body)


def _expand_body(tok_ref, table_ref, alias_ref, out_ref):
    del alias_ref
    tok = tok_ref[0]
    oh = (tok[0, :, None] == lax.broadcasted_iota(jnp.int32, (1, _V), 1)
          ).astype(jnp.float32)
    out_ref[:] = jax.lax.dot_general(
        oh, table_ref[:],
        dimension_numbers=(((1,), (0,)), ((), ())),
        preferred_element_type=jnp.float32)


_expand = pl.pallas_call(
    _expand_body,
    grid=((_B - _S) // _R,),
    in_specs=[
        pl.BlockSpec((1, 1, _R), lambda i: (i, 0, 0)),
        pl.BlockSpec((_V, _D), lambda i: (0, 0)),
        pl.BlockSpec(memory_space=pl.ANY),
    ],
    out_specs=pl.BlockSpec((_R, _D), lambda i: (_S // _R + i, 0)),
    out_shape=jax.ShapeDtypeStruct((_B, _D), jnp.float32),
    input_output_aliases={2: 0},
)


def kernel(tokens, emb, W, b):
    table = _compute_table(emb, W, b)
    tok = tokens.astype(jnp.int32).reshape(-1)
    tok_lo = tok[:_S].reshape(_NW, _NCHUNK, _CHUNK)
    tok_hi = tok[_S:].reshape((_B - _S) // _R, 1, _R)
    partial = _gather(tok_lo, table)
    out = _expand(tok_hi, table, partial)
    return out.reshape(tokens.shape[0], tokens.shape[1], _D)
